# Initial kernel scaffold; baseline (speedup 1.0000x reference)
import jax
import jax.numpy as jnp
from jax.experimental import pallas as pl


def kernel(input_ids, emb_table, W, b):
    raise NotImplementedError("write your pallas kernel here")


# speedup vs baseline: 2.9078x; 2.9078x over previous
import functools

import jax
import jax.numpy as jnp
from jax import lax
from jax.experimental import pallas as pl
from jax.experimental.pallas import tpu as pltpu
from jax.experimental.pallas import tpu_sc as plsc

VOCAB = 100
EMB = 8
TOK = 16384 * 200

NC, NS = 2, 16
NW = NC * NS
PER_W = TOK // NW
BATCH = 128
CHUNK = 16
BLK = BATCH * CHUNK
NBLK = PER_W // BLK


def _fuse_table_body(emb_ref, w_ref, b_ref, out_ref):
    out_ref[...] = lax.dot_general(
        emb_ref[...], w_ref[...],
        dimension_numbers=(((1,), (1,)), ((), ())),
        preferred_element_type=jnp.float32) + b_ref[...]


def _fused_table(emb_table, W, b):
    return pl.pallas_call(
        _fuse_table_body,
        out_shape=jax.ShapeDtypeStruct((VOCAB, EMB), jnp.float32),
    )(emb_table, W, b.reshape(1, EMB))


_sc_mesh = plsc.VectorSubcoreMesh(core_axis_name="c", subcore_axis_name="s")


@functools.partial(
    pl.kernel,
    out_type=jax.ShapeDtypeStruct((TOK, EMB), jnp.float32),
    mesh=_sc_mesh,
    scratch_types=[
        pltpu.VMEM((CHUNK, BATCH), jnp.int32),
        pltpu.VMEM((BLK, EMB), jnp.float32),
        pltpu.SemaphoreType.DMA,
    ],
    compiler_params=pltpu.CompilerParams(use_tc_tiling_on_sc=False),
)
def _gather_kernel(table_hbm, ids_hbm, out_hbm, idx_v, rows_v, gsem):
    wid = lax.axis_index("s") * NC + lax.axis_index("c")
    idx_row0 = wid * (PER_W // BATCH)

    def body(blk, _):
        pltpu.sync_copy(ids_hbm.at[pl.ds(idx_row0 + blk * CHUNK, CHUNK)], idx_v)
        descs = [
            pltpu.async_copy(
                table_hbm.at[idx_v.at[j]],
                rows_v.at[pl.ds(j * BATCH, BATCH)],
                gsem,
            )
            for j in range(CHUNK)
        ]
        for d in descs:
            d.wait()
        pltpu.sync_copy(rows_v, out_hbm.at[pl.ds(wid * PER_W + blk * BLK, BLK)])
        return ()

    lax.fori_loop(0, NBLK, body, (), unroll=False)


def kernel(input_ids, emb_table, W, b):
    table = _fused_table(emb_table, W, b)
    ids = input_ids.reshape(TOK // BATCH, BATCH).astype(jnp.int32)
    out = _gather_kernel(table, ids)
    return out.reshape(input_ids.shape[0], input_ids.shape[1], EMB)


# speedup vs baseline: 5.6260x; 1.9348x over previous
import functools

import jax
import jax.numpy as jnp
from jax import lax
from jax.experimental import pallas as pl
from jax.experimental.pallas import tpu as pltpu
from jax.experimental.pallas import tpu_sc as plsc

VOCAB = 100
EMB = 8
TOK = 16384 * 200

NC, NS = 2, 16
NW = NC * NS
PER_W = TOK // NW
GRP = 16
BLK = 2048
NBLK = PER_W // BLK
GPB = BLK // GRP


def _fuse_table_body(emb_ref, w_ref, b_ref, out_ref):
    out_ref[...] = lax.dot_general(
        emb_ref[...], w_ref[...],
        dimension_numbers=(((1,), (1,)), ((), ())),
        preferred_element_type=jnp.float32) + b_ref[...]


def _fused_table(emb_table, W, b):
    return pl.pallas_call(
        _fuse_table_body,
        out_shape=jax.ShapeDtypeStruct((VOCAB, EMB), jnp.float32),
    )(emb_table, W, b.reshape(1, EMB))


_sc_mesh = plsc.VectorSubcoreMesh(core_axis_name="c", subcore_axis_name="s")


@functools.partial(
    pl.kernel,
    out_type=jax.ShapeDtypeStruct((TOK * EMB,), jnp.float32),
    mesh=_sc_mesh,
    scratch_types=[
        pltpu.VMEM((VOCAB * EMB,), jnp.float32),
        pltpu.VMEM((PER_W,), jnp.int32),
        pltpu.VMEM((BLK * EMB,), jnp.float32),
    ],
    compiler_params=pltpu.CompilerParams(
        use_tc_tiling_on_sc=False, needs_layout_passes=False),
)
def _gather_kernel(table_hbm, ids_hbm, out_hbm, table_v, ids_v, out_v):
    wid = lax.axis_index("s") * NC + lax.axis_index("c")
    tok0 = wid * PER_W

    pltpu.sync_copy(table_hbm, table_v)
    pltpu.sync_copy(ids_hbm.at[pl.ds(tok0, PER_W)], ids_v)

    iota = lax.iota(jnp.int32, GRP)
    cols = [iota * EMB + j for j in range(EMB)]

    def group(gg, g, _):
        ids = ids_v[pl.ds(gg * GRP, GRP)]
        flat = ids * EMB
        ob = out_v.at[pl.ds(g * (GRP * EMB), GRP * EMB)]
        for j in range(EMB):
            col = plsc.load_gather(table_v, [flat + j])
            plsc.store_scatter(ob, [cols[j]], col)
        return ()

    def block(blk, _):
        lax.fori_loop(0, GPB,
                      lambda g, c: group(blk * GPB + g, g, c), (),
                      unroll=4)
        pltpu.sync_copy(
            out_v, out_hbm.at[pl.ds((tok0 + blk * BLK) * EMB, BLK * EMB)])
        return ()

    lax.fori_loop(0, NBLK, block, ())


def kernel(input_ids, emb_table, W, b):
    table = _fused_table(emb_table, W, b).reshape(VOCAB * EMB)
    ids = input_ids.reshape(TOK).astype(jnp.int32)
    out = _gather_kernel(table, ids)
    return out.reshape(input_ids.shape[0], input_ids.shape[1], EMB)


# speedup vs baseline: 6.0493x; 1.0752x over previous
import functools

import jax
import jax.numpy as jnp
from jax import lax
from jax.experimental import pallas as pl
from jax.experimental.pallas import tpu as pltpu
from jax.experimental.pallas import tpu_sc as plsc

VOCAB = 100
EMB = 8
TOK = 16384 * 200

NC, NS = 2, 16
NW = NC * NS
PER_W = TOK // NW
GRP = 16
BLK = 800
NBLK = PER_W // BLK
GPB = BLK // GRP


def _fuse_table_body(emb_ref, w_ref, b_ref, out_ref):
    out_ref[...] = lax.dot_general(
        emb_ref[...], w_ref[...],
        dimension_numbers=(((1,), (1,)), ((), ())),
        preferred_element_type=jnp.float32) + b_ref[...]


def _fused_table(emb_table, W, b):
    return pl.pallas_call(
        _fuse_table_body,
        out_shape=jax.ShapeDtypeStruct((VOCAB, EMB), jnp.float32),
    )(emb_table, W, b.reshape(1, EMB))


_sc_mesh = plsc.VectorSubcoreMesh(core_axis_name="c", subcore_axis_name="s")


@functools.partial(
    pl.kernel,
    out_type=jax.ShapeDtypeStruct((TOK, EMB), jnp.float32),
    mesh=_sc_mesh,
    scratch_types=[
        pltpu.VMEM((VOCAB * EMB,), jnp.float32),
        pltpu.VMEM((BLK,), jnp.int32),
        pltpu.VMEM((BLK, EMB), jnp.float32),
    ],
    compiler_params=pltpu.CompilerParams(needs_layout_passes=False),
)
def _gather_kernel(table_hbm, ids_hbm, out_hbm, table_v, ids_v, out_v):
    wid = lax.axis_index("s") * NC + lax.axis_index("c")
    tok0 = wid * PER_W

    pltpu.sync_copy(table_hbm, table_v)

    iota = lax.iota(jnp.int32, GRP)
    cols = [jnp.full((GRP,), j, jnp.int32) for j in range(EMB)]

    def group(g, _):
        ids = ids_v[pl.ds(g * GRP, GRP)]
        flat = ids * EMB
        rows = iota + g * GRP
        for j in range(EMB):
            col = plsc.load_gather(table_v, [flat + j])
            plsc.store_scatter(out_v, [rows, cols[j]], col)
        return ()

    def block(blk, _):
        pltpu.sync_copy(ids_hbm.at[pl.ds(tok0 + blk * BLK, BLK)], ids_v)
        lax.fori_loop(0, GPB, group, (), unroll=4)
        pltpu.sync_copy(out_v, out_hbm.at[pl.ds(tok0 + blk * BLK, BLK), :])
        return ()

    lax.fori_loop(0, NBLK, block, ())


def kernel(input_ids, emb_table, W, b):
    table = _fused_table(emb_table, W, b).reshape(VOCAB * EMB)
    ids = input_ids.reshape(TOK).astype(jnp.int32)
    out = _gather_kernel(table, ids)
    return out.reshape(input_ids.shape[0], input_ids.shape[1], EMB)
